# baseline (speedup 1.0000x reference)
import jax
import jax.numpy as jnp
from jax.experimental import pallas as pl
from jax.experimental.pallas import tpu as pltpu

NUM_CAND = 1000
MAX_DETS = 100
NMS_THR = 0.5
NUM_CLASSES = 80
NUM_FEATS = 20000
ROWS = (NUM_FEATS * NUM_CLASSES) // 128
ROWS_PAD = 12800
CH = 64
NCHUNK = ROWS_PAD // CH
BIG = 1 << 30
NEG = float("-inf")
ONE_BITS = 0x3F800000


def _iota(shape, dim):
    return jax.lax.broadcasted_iota(jnp.int32, shape, dim)


def _body(logits_ref, boxes_ref, lab_out, box_out, sc_out,
          sig, scoreC, labC, rows_c, sc2d, lab2d,
          bx1, by1, bx2, by2, areas, packed,
          pg_s, pe_s, cg_s, ce_s):
    flat8 = _iota((8, 128), 0) * 128 + _iota((8, 128), 1)
    lane128 = _iota((1, 128), 1)
    flat_ch = _iota((CH, 128), 0) * 128 + _iota((CH, 128), 1)
    row_ch = _iota((CH, 128), 0)

    def p1(c, _):
        lg = logits_ref[pl.ds(c * CH, CH), :]
        s = jax.nn.sigmoid(lg)
        sig[pl.ds(c * CH, CH), :] = jnp.where(row_ch + c * CH < ROWS, s, NEG)
        return 0

    jax.lax.fori_loop(0, NCHUNK, p1, 0)

    def count_gt(tm):
        def cb(b, acc):
            blk = sig[pl.ds(b * 1600, 1600), :]
            return acc + jnp.sum((blk > tm).astype(jnp.int32))
        return jax.lax.fori_loop(0, 8, cb, jnp.int32(0))

    def bs(_, lohi):
        lo, hi = lohi
        mid = (lo + hi) // 2
        tm = jax.lax.bitcast_convert_type(mid, jnp.float32)
        big = count_gt(tm) >= NUM_CAND
        return (jnp.where(big, mid, lo), jnp.where(big, hi, mid))

    _, hi = jax.lax.fori_loop(
        0, 30, bs, (jnp.int32(0), jnp.int32(ONE_BITS)))
    V = jax.lax.bitcast_convert_type(hi, jnp.float32)

    def pt(c, carry):
        pg, pe = carry
        blk = sig[pl.ds(c * CH, CH), :]
        cg = jnp.sum((blk > V).astype(jnp.int32))
        ceq = jnp.sum((blk == V).astype(jnp.int32))
        pg_s[c] = pg
        pe_s[c] = pe
        cg_s[c] = cg
        ce_s[c] = ceq
        return (pg + cg, pe + ceq)

    G, _ = jax.lax.fori_loop(0, NCHUNK, pt, (jnp.int32(0), jnp.int32(0)))
    q = NUM_CAND - G

    scoreC[:, :] = jnp.full((1024, 1), NEG)
    labC[:, :] = jnp.zeros((1024, 1), jnp.int32)
    rows_c[:, :] = jnp.zeros((1024, 4), jnp.float32)

    def ex(c, k):
        cg = cg_s[c]
        ceq = ce_s[c]
        pe = pe_s[c]
        nk_eq = jnp.clip(q - pe, 0, ceq)
        blk = sig[pl.ds(c * CH, CH), :]
        eqc = jnp.where(blk == V, flat_ch, -1)

        def drop(_, e):
            pmax = jnp.max(e)
            return jnp.where(flat_ch == pmax, -1, e)

        trips = jnp.where(nk_eq > 0, ceq - nk_eq, 0)
        eqc = jax.lax.fori_loop(0, trips, drop, eqc)
        cand0 = jnp.where(blk > V, flat_ch, BIG)
        cand0 = jnp.where((eqc >= 0) & (nk_eq > 0), flat_ch, cand0)

        def pull(_, st):
            cand, kk = st
            p = jnp.min(cand)
            rr = p // 128
            s_row = sig[pl.ds(c * CH + rr, 1), :]
            s = jnp.max(jnp.where(lane128 == p - rr * 128, s_row, NEG))
            fl = c * (CH * 128) + p
            feat = fl // NUM_CLASSES
            lab = fl - feat * NUM_CLASSES
            scoreC[pl.ds(kk, 1), :] = s.reshape(1, 1)
            labC[pl.ds(kk, 1), :] = lab.reshape(1, 1)
            rows_c[pl.ds(kk, 1), :] = boxes_ref[pl.ds(feat, 1), :]
            return (jnp.where(flat_ch == p, BIG, cand), kk + 1)

        _, k2 = jax.lax.fori_loop(0, cg + nk_eq, pull, (cand0, k))
        return k2

    jax.lax.fori_loop(0, NCHUNK, ex, jnp.int32(0))

    def rl(b, _):
        sc2d[pl.ds(b, 1), :] = scoreC[pl.ds(b * 128, 128), :].reshape(1, 128)
        lbB = labC[pl.ds(b * 128, 128), :].astype(jnp.float32)
        lab2d[pl.ds(b, 1), :] = lbB.reshape(1, 128)
        rB = rows_c[pl.ds(b * 128, 128), :]
        bx1[pl.ds(b, 1), :] = rB[:, 0:1].reshape(1, 128)
        by1[pl.ds(b, 1), :] = rB[:, 1:2].reshape(1, 128)
        bx2[pl.ds(b, 1), :] = rB[:, 2:3].reshape(1, 128)
        by2[pl.ds(b, 1), :] = rB[:, 3:4].reshape(1, 128)
        return 0

    jax.lax.fori_loop(0, 8, rl, 0)

    valid8 = flat8 < NUM_CAND
    mc = jnp.max(jnp.where(
        valid8,
        jnp.maximum(jnp.maximum(bx1[:, :], by1[:, :]),
                    jnp.maximum(bx2[:, :], by2[:, :])),
        NEG)) + 1.0
    lab_f = lab2d[:, :]
    bx1[:, :] = bx1[:, :] + lab_f * mc
    by1[:, :] = by1[:, :] + lab_f * mc
    bx2[:, :] = bx2[:, :] + lab_f * mc
    by2[:, :] = by2[:, :] + lab_f * mc
    areas[:, :] = (bx2[:, :] - bx1[:, :]) * (by2[:, :] - by1[:, :])

    def p3(b, _):
        rows = rows_c[pl.ds(b * 128, 128), :]
        labf = labC[pl.ds(b * 128, 128), :].astype(jnp.float32)
        orows = rows + labf * mc
        packed[pl.ds(b * 128, 128), 0:4] = orows
        packed[pl.ds(b * 128, 128), 4:8] = rows
        return 0

    jax.lax.fori_loop(0, 8, p3, 0)

    act0 = sc2d[:, :]
    m0 = jnp.max(act0)
    i0 = jnp.min(jnp.where(act0 == m0, flat8, BIG))

    def nms(t, act):
        m = jnp.max(act)
        i_raw = jnp.min(jnp.where(act == m, flat8, BIG))
        valid = m != NEG
        i = jnp.where(valid, i_raw, i0)
        pk = packed[pl.ds(i, 1), :]
        x1i = pk[:, 0:1]
        y1i = pk[:, 1:2]
        x2i = pk[:, 2:3]
        y2i = pk[:, 3:4]
        area_i = (x2i - x1i) * (y2i - y1i)
        ix1 = jnp.maximum(x1i, bx1[:, :])
        iy1 = jnp.maximum(y1i, by1[:, :])
        ix2 = jnp.minimum(x2i, bx2[:, :])
        iy2 = jnp.minimum(y2i, by2[:, :])
        inter = jnp.maximum(ix2 - ix1, 0.0) * jnp.maximum(iy2 - iy1, 0.0)
        iou = inter / (area_i + areas[:, :] - inter + 1e-9)
        act2 = jnp.where(iou > NMS_THR, NEG, act)
        act2 = jnp.where(flat8 == i, NEG, act2)
        sc_out[pl.ds(t, 1), :] = jnp.where(valid, m, 0.0).reshape(1, 1)
        li = labC[pl.ds(i, 1), :]
        lab_out[pl.ds(t, 1), :] = jnp.where(valid, li, -1)
        box_out[pl.ds(t, 1), :] = pk[:, 4:8]
        return act2

    jax.lax.fori_loop(0, MAX_DETS, nms, act0)


@jax.jit
def kernel(cls_logits, boxes):
    sc = cls_logits[:, :NUM_CLASSES].reshape(ROWS, 128)
    sc = jnp.pad(sc, ((0, ROWS_PAD - ROWS), (0, 0)))
    labs, bxs, scs = pl.pallas_call(
        _body,
        out_shape=[
            jax.ShapeDtypeStruct((MAX_DETS, 1), jnp.int32),
            jax.ShapeDtypeStruct((MAX_DETS, 4), jnp.float32),
            jax.ShapeDtypeStruct((MAX_DETS, 1), jnp.float32),
        ],
        scratch_shapes=[
            pltpu.VMEM((ROWS_PAD, 128), jnp.float32),
            pltpu.VMEM((1024, 1), jnp.float32),
            pltpu.VMEM((1024, 1), jnp.int32),
            pltpu.VMEM((1024, 4), jnp.float32),
            pltpu.VMEM((8, 128), jnp.float32),
            pltpu.VMEM((8, 128), jnp.float32),
            pltpu.VMEM((8, 128), jnp.float32),
            pltpu.VMEM((8, 128), jnp.float32),
            pltpu.VMEM((8, 128), jnp.float32),
            pltpu.VMEM((8, 128), jnp.float32),
            pltpu.VMEM((8, 128), jnp.float32),
            pltpu.VMEM((1024, 8), jnp.float32),
            pltpu.SMEM((256,), jnp.int32),
            pltpu.SMEM((256,), jnp.int32),
            pltpu.SMEM((256,), jnp.int32),
            pltpu.SMEM((256,), jnp.int32),
        ],
    )(sc, boxes)
    return labs.reshape(MAX_DETS), bxs, scs.reshape(MAX_DETS)


# speedup vs baseline: 1.2308x; 1.2308x over previous
import jax
import jax.numpy as jnp
from jax.experimental import pallas as pl
from jax.experimental.pallas import tpu as pltpu

NUM_CAND = 1000
MAX_DETS = 100
NMS_THR = 0.5
NUM_CLASSES = 80
NUM_FEATS = 20000
ROWS = (NUM_FEATS * NUM_CLASSES) // 128
ROWS_PAD = 12800
CH = 64
NCHUNK = ROWS_PAD // CH
BIG = 1 << 30
NEG = float("-inf")
ONE_BITS = 0x3F800000


def _iota(shape, dim):
    return jax.lax.broadcasted_iota(jnp.int32, shape, dim)


def _body(logits_ref, boxes_ref, lab_out, box_out, sc_out,
          sig, scoreC, labC, rows_c, sc2d, lab2d,
          bx1, by1, bx2, by2, areas, packed,
          pg_s, pe_s, cg_s, ce_s):
    flat8 = _iota((8, 128), 0) * 128 + _iota((8, 128), 1)
    lane128 = _iota((1, 128), 1)
    flat_ch = _iota((CH, 128), 0) * 128 + _iota((CH, 128), 1)
    row_ch = _iota((CH, 128), 0)

    def p1(c, _):
        lg = logits_ref[pl.ds(c * CH, CH), :]
        s = jax.nn.sigmoid(lg)
        sig[pl.ds(c * CH, CH), :] = jnp.where(row_ch + c * CH < ROWS, s, NEG)
        return 0

    jax.lax.fori_loop(0, NCHUNK, p1, 0)

    def count_gt(tm):
        def cb(b, acc):
            blk = sig[pl.ds(b * 1600, 1600), :]
            return acc + jnp.sum((blk > tm).astype(jnp.int32))
        return jax.lax.fori_loop(0, 8, cb, jnp.int32(0))

    def bs(_, lohi):
        lo, hi = lohi
        mid = (lo + hi) // 2
        tm = jax.lax.bitcast_convert_type(mid, jnp.float32)
        big = count_gt(tm) >= NUM_CAND
        return (jnp.where(big, mid, lo), jnp.where(big, hi, mid))

    _, hi = jax.lax.fori_loop(
        0, 30, bs, (jnp.int32(0), jnp.int32(ONE_BITS)))
    V = jax.lax.bitcast_convert_type(hi, jnp.float32)

    def pt(c, carry):
        pg, pe = carry
        blk = sig[pl.ds(c * CH, CH), :]
        cg = jnp.sum((blk > V).astype(jnp.int32))
        ceq = jnp.sum((blk == V).astype(jnp.int32))
        pg_s[c] = pg
        pe_s[c] = pe
        cg_s[c] = cg
        ce_s[c] = ceq
        return (pg + cg, pe + ceq)

    G, _ = jax.lax.fori_loop(0, NCHUNK, pt, (jnp.int32(0), jnp.int32(0)))
    q = NUM_CAND - G

    scoreC[:, :] = jnp.full((1024, 128), NEG)
    labC[:, :] = jnp.zeros((1024, 1), jnp.int32)
    rows_c[:, :] = jnp.zeros((1024, 4), jnp.float32)

    def ex(c, k):
        cg = cg_s[c]
        ceq = ce_s[c]
        pe = pe_s[c]
        nk_eq = jnp.clip(q - pe, 0, ceq)
        blk = sig[pl.ds(c * CH, CH), :]
        eqc = jnp.where(blk == V, flat_ch, -1)

        def drop(_, e):
            pmax = jnp.max(e)
            return jnp.where(flat_ch == pmax, -1, e)

        trips = jnp.where(nk_eq > 0, ceq - nk_eq, 0)
        eqc = jax.lax.fori_loop(0, trips, drop, eqc)
        cand0 = jnp.where(blk > V, flat_ch, BIG)
        cand0 = jnp.where((eqc >= 0) & (nk_eq > 0), flat_ch, cand0)

        def pull(_, st):
            cand, kk = st
            p = jnp.min(cand)
            rr = p // 128
            s_row = sig[pl.ds(c * CH + rr, 1), :]
            fl = c * (CH * 128) + p
            feat = fl // NUM_CLASSES
            lab = fl - feat * NUM_CLASSES
            scoreC[pl.ds(kk, 1), :] = jnp.where(
                lane128 == p - rr * 128, s_row, NEG)
            labC[pl.ds(kk, 1), :] = lab.reshape(1, 1)
            rows_c[pl.ds(kk, 1), :] = boxes_ref[pl.ds(feat, 1), :]
            return (jnp.where(flat_ch == p, BIG, cand), kk + 1)

        _, k2 = jax.lax.fori_loop(0, cg + nk_eq, pull, (cand0, k))
        return k2

    jax.lax.fori_loop(0, NCHUNK, ex, jnp.int32(0))

    def rl(b, _):
        blkS = scoreC[pl.ds(b * 128, 128), :]
        sc2d[pl.ds(b, 1), :] = jnp.max(blkS, axis=1, keepdims=True).reshape(1, 128)
        lbB = labC[pl.ds(b * 128, 128), :].astype(jnp.float32)
        lab2d[pl.ds(b, 1), :] = lbB.reshape(1, 128)
        rB = rows_c[pl.ds(b * 128, 128), :]
        bx1[pl.ds(b, 1), :] = rB[:, 0:1].reshape(1, 128)
        by1[pl.ds(b, 1), :] = rB[:, 1:2].reshape(1, 128)
        bx2[pl.ds(b, 1), :] = rB[:, 2:3].reshape(1, 128)
        by2[pl.ds(b, 1), :] = rB[:, 3:4].reshape(1, 128)
        return 0

    jax.lax.fori_loop(0, 8, rl, 0)

    valid8 = flat8 < NUM_CAND
    mc = jnp.max(jnp.where(
        valid8,
        jnp.maximum(jnp.maximum(bx1[:, :], by1[:, :]),
                    jnp.maximum(bx2[:, :], by2[:, :])),
        NEG)) + 1.0
    lab_f = lab2d[:, :]
    bx1[:, :] = bx1[:, :] + lab_f * mc
    by1[:, :] = by1[:, :] + lab_f * mc
    bx2[:, :] = bx2[:, :] + lab_f * mc
    by2[:, :] = by2[:, :] + lab_f * mc
    areas[:, :] = (bx2[:, :] - bx1[:, :]) * (by2[:, :] - by1[:, :])

    def p3(b, _):
        rows = rows_c[pl.ds(b * 128, 128), :]
        labf = labC[pl.ds(b * 128, 128), :].astype(jnp.float32)
        orows = rows + labf * mc
        packed[pl.ds(b * 128, 128), 0:4] = orows
        packed[pl.ds(b * 128, 128), 4:8] = rows
        return 0

    jax.lax.fori_loop(0, 8, p3, 0)

    act0 = sc2d[:, :]
    m0 = jnp.max(act0)
    i0 = jnp.min(jnp.where(act0 == m0, flat8, BIG))

    def nms(t, act):
        m = jnp.max(act)
        i_raw = jnp.min(jnp.where(act == m, flat8, BIG))
        valid = m != NEG
        i = jnp.where(valid, i_raw, i0)
        pk = packed[pl.ds(i, 1), :]
        x1i = pk[:, 0:1]
        y1i = pk[:, 1:2]
        x2i = pk[:, 2:3]
        y2i = pk[:, 3:4]
        area_i = (x2i - x1i) * (y2i - y1i)
        ix1 = jnp.maximum(x1i, bx1[:, :])
        iy1 = jnp.maximum(y1i, by1[:, :])
        ix2 = jnp.minimum(x2i, bx2[:, :])
        iy2 = jnp.minimum(y2i, by2[:, :])
        inter = jnp.maximum(ix2 - ix1, 0.0) * jnp.maximum(iy2 - iy1, 0.0)
        iou = inter / (area_i + areas[:, :] - inter + 1e-9)
        act2 = jnp.where(iou > NMS_THR, NEG, act)
        act2 = jnp.where(flat8 == i, NEG, act2)
        sc_out[pl.ds(t, 1), :] = jnp.where(valid, m, 0.0).reshape(1, 1)
        li = labC[pl.ds(i, 1), :]
        lab_out[pl.ds(t, 1), :] = jnp.where(valid, li, -1)
        box_out[pl.ds(t, 1), :] = pk[:, 4:8]
        return act2

    jax.lax.fori_loop(0, MAX_DETS, nms, act0)


@jax.jit
def kernel(cls_logits, boxes):
    sc = cls_logits[:, :NUM_CLASSES].reshape(ROWS, 128)
    sc = jnp.pad(sc, ((0, ROWS_PAD - ROWS), (0, 0)))
    labs, bxs, scs = pl.pallas_call(
        _body,
        out_shape=[
            jax.ShapeDtypeStruct((MAX_DETS, 1), jnp.int32),
            jax.ShapeDtypeStruct((MAX_DETS, 4), jnp.float32),
            jax.ShapeDtypeStruct((MAX_DETS, 1), jnp.float32),
        ],
        scratch_shapes=[
            pltpu.VMEM((ROWS_PAD, 128), jnp.float32),
            pltpu.VMEM((1024, 128), jnp.float32),
            pltpu.VMEM((1024, 1), jnp.int32),
            pltpu.VMEM((1024, 4), jnp.float32),
            pltpu.VMEM((8, 128), jnp.float32),
            pltpu.VMEM((8, 128), jnp.float32),
            pltpu.VMEM((8, 128), jnp.float32),
            pltpu.VMEM((8, 128), jnp.float32),
            pltpu.VMEM((8, 128), jnp.float32),
            pltpu.VMEM((8, 128), jnp.float32),
            pltpu.VMEM((8, 128), jnp.float32),
            pltpu.VMEM((1024, 8), jnp.float32),
            pltpu.SMEM((256,), jnp.int32),
            pltpu.SMEM((256,), jnp.int32),
            pltpu.SMEM((256,), jnp.int32),
            pltpu.SMEM((256,), jnp.int32),
        ],
    )(sc, boxes)
    return labs.reshape(MAX_DETS), bxs, scs.reshape(MAX_DETS)
